# baseline (speedup 1.0000x reference)
import functools

import jax
import jax.numpy as jnp
from jax import lax
from jax.experimental import pallas as pl
from jax.experimental.pallas import tpu as pltpu
from jax.experimental.pallas import tpu_sc as plsc

QUEUE = 16384
FEAT = 64 * 7 * 7
BATCH = 4096
NC, NS, L = 2, 16, 16
NW = NC * NS
ROWS_PW = QUEUE // NW
NVREG = BATCH // L
CHUNK = 16
NCHUNK = ROWS_PW // CHUNK
CAP = ROWS_PW + 2 * L
NSLOT = 16


def _sc_scatter_body(batch_hbm, idx_hbm, feat_hbm, out_hbm,
                     idx_v, winner_v, kpos_v, kdst_v, buf_v):
    wid = lax.axis_index("s") * NC + lax.axis_index("c")
    lo = wid * ROWS_PW
    iota = lax.iota(jnp.int32, L)

    pltpu.sync_copy(idx_hbm, idx_v)

    def scan1(g, carry):
        x = idx_v[pl.ds(g * L, L)]
        posv = jnp.full((L,), g * L, jnp.int32) + iota
        keep = posv >= 0
        for s in range(1, L):
            sh = jnp.take_along_axis(x, jnp.minimum(iota + s, L - 1), axis=0)
            dup = (sh == x) & (iota < (L - s))
            keep = keep & (~dup)
        xl = x - lo
        in_range = (xl >= 0) & (xl < ROWS_PW)
        xl = jnp.clip(xl, 0, ROWS_PW - 1)
        plsc.store_scatter(winner_v, [xl], posv, mask=keep & in_range)
        return carry

    lax.fori_loop(0, NVREG, scan1, 0)

    def scan2(g, off):
        x = idx_v[pl.ds(g * L, L)]
        posv = jnp.full((L,), g * L, jnp.int32) + iota
        xl = x - lo
        in_range = (xl >= 0) & (xl < ROWS_PW)
        xl = jnp.clip(xl, 0, ROWS_PW - 1)
        w = plsc.load_gather(winner_v, [xl])
        m = in_range & (w == posv)
        mi = m.astype(jnp.int32)
        rank = jnp.full((L,), off, jnp.int32) + lax.cumsum(mi, axis=0) - 1
        plsc.store_scatter(kpos_v, [rank], posv, mask=m)
        plsc.store_scatter(kdst_v, [rank], x, mask=m)
        return off + jnp.sum(mi)

    cnt = lax.fori_loop(0, NVREG, scan2, jnp.int32(0))

    rem = lax.rem(cnt, jnp.int32(NSLOT))

    @pl.when(rem != 0)
    def _pad():
        zero = jnp.zeros((L,), jnp.int32)
        p0 = plsc.load_gather(kpos_v, [zero])
        d0 = plsc.load_gather(kdst_v, [zero])
        base = cnt - rem
        curp = kpos_v[pl.ds(base, L)]
        curd = kdst_v[pl.ds(base, L)]
        msk = iota < rem
        kpos_v[pl.ds(base, L)] = jnp.where(msk, curp, p0)
        kdst_v[pl.ds(base, L)] = jnp.where(msk, curd, d0)

    ngroups = lax.div(cnt + jnp.int32(NSLOT - 1), jnp.int32(NSLOT))

    cw = CHUNK * FEAT

    def cbody(t, carry):
        base = (lo + t * CHUNK) * FEAT
        pltpu.sync_copy(feat_hbm.at[pl.ds(base, cw)], buf_v.at[pl.ds(0, cw)])
        pltpu.sync_copy(buf_v.at[pl.ds(0, cw)], out_hbm.at[pl.ds(base, cw)])
        return carry

    lax.fori_loop(0, NCHUNK, cbody, 0)

    def _slot(s):
        return buf_v.at[pl.ds(s * FEAT, FEAT)]

    def sbody(j, carry):
        pvec = kpos_v[pl.ds(j * NSLOT, L)]
        dvec = kdst_v[pl.ds(j * NSLOT, L)]
        for s in range(NSLOT):
            pos = jnp.sum(jnp.where(iota == s, pvec, jnp.int32(0)))
            pltpu.sync_copy(batch_hbm.at[pl.ds(pos * FEAT, FEAT)], _slot(s))
        for s in range(NSLOT):
            dst = jnp.sum(jnp.where(iota == s, dvec, jnp.int32(0)))
            pltpu.sync_copy(_slot(s), out_hbm.at[pl.ds(dst * FEAT, FEAT)])
        return carry

    lax.fori_loop(0, ngroups, sbody, 0)


_sc_scatter = functools.partial(
    pl.kernel,
    out_type=jax.ShapeDtypeStruct((QUEUE * FEAT,), jnp.float32),
    mesh=plsc.VectorSubcoreMesh(core_axis_name="c", subcore_axis_name="s"),
    compiler_params=pltpu.CompilerParams(needs_layout_passes=False),
    scratch_types=[
        pltpu.VMEM((BATCH,), jnp.int32),
        pltpu.VMEM((ROWS_PW,), jnp.int32),
        pltpu.VMEM((CAP,), jnp.int32),
        pltpu.VMEM((CAP,), jnp.int32),
        pltpu.VMEM((2 * CHUNK * FEAT,), jnp.float32),
    ],
)(_sc_scatter_body)


def kernel(batch_features, batch_indices, features):
    bf = batch_features.reshape(BATCH * FEAT)
    ft = features.reshape(QUEUE * FEAT)
    out = _sc_scatter(bf, batch_indices, ft)
    return out.reshape(QUEUE, 64, 7, 7)


# speedup vs baseline: 22.0111x; 22.0111x over previous
import functools

import jax
import jax.numpy as jnp
from jax import lax
from jax.experimental import pallas as pl
from jax.experimental.pallas import tpu as pltpu
from jax.experimental.pallas import tpu_sc as plsc

QUEUE = 16384
BATCH = 4096
NC, NS, L = 2, 16, 16
NW = NC * NS
NVREG = BATCH // L
G = 7 * 7 * 8
QT = QUEUE // 128
PT = BATCH // 128
HALF = QT // 2
CAP = 4096 + 2 * L
NG_PER = (G + NW - 1) // NW


def _sc_body(batch_hbm, idx_hbm, feat_hbm, out_hbm,
             idx_v, winner_v, hpos_v, hdst_v, blk_v, bfb_v):
    wid = lax.axis_index("s") * NC + lax.axis_index("c")
    iota = lax.iota(jnp.int32, L)
    zero = jnp.zeros((L,), jnp.int32)

    pltpu.sync_copy(idx_hbm, idx_v)

    def scan1(g, carry):
        x = idx_v[pl.ds(g * L, L)]
        posv = jnp.full((L,), g * L, jnp.int32) + iota
        keep = posv >= 0
        for s in range(1, L):
            sh = jnp.take_along_axis(x, jnp.minimum(iota + s, L - 1), axis=0)
            dup = (sh == x) & (iota < (L - s))
            keep = keep & (~dup)
        plsc.store_scatter(winner_v, [x], posv, mask=keep)
        return carry

    lax.fori_loop(0, NVREG, scan1, 0)

    def scan2(g, offs):
        off0, off1 = offs
        x = idx_v[pl.ds(g * L, L)]
        posv = jnp.full((L,), g * L, jnp.int32) + iota
        w = plsc.load_gather(winner_v, [x])
        m = w == posv
        dt = jnp.right_shift(x, 7)
        m0 = m & (dt < HALF)
        m1 = m & (dt >= HALF)
        c0 = lax.cumsum(m0.astype(jnp.int32), axis=0)
        c1 = lax.cumsum(m1.astype(jnp.int32), axis=0)
        rank0 = jnp.full((L,), off0, jnp.int32) + c0 - 1
        rank1 = jnp.full((L,), CAP - off1, jnp.int32) - c1
        plsc.store_scatter(hpos_v, [rank0], posv, mask=m0)
        plsc.store_scatter(hdst_v, [rank0], x, mask=m0)
        plsc.store_scatter(hpos_v, [rank1], posv, mask=m1)
        plsc.store_scatter(hdst_v, [rank1], x, mask=m1)
        return (off0 + jnp.sum(m0.astype(jnp.int32)),
                off1 + jnp.sum(m1.astype(jnp.int32)))

    cnt0, cnt1 = lax.fori_loop(0, NVREG, scan2, (jnp.int32(0), jnp.int32(0)))

    rem0 = lax.rem(cnt0, jnp.int32(L))

    @pl.when(rem0 != 0)
    def _pad0():
        p0 = plsc.load_gather(hpos_v, [zero])
        d0 = plsc.load_gather(hdst_v, [zero])
        base = cnt0 - rem0
        msk = iota < rem0
        hpos_v[pl.ds(base, L)] = jnp.where(msk, hpos_v[pl.ds(base, L)], p0)
        hdst_v[pl.ds(base, L)] = jnp.where(msk, hdst_v[pl.ds(base, L)], d0)

    rem1 = lax.rem(cnt1, jnp.int32(L))

    @pl.when(rem1 != 0)
    def _pad1():
        top = jnp.full((L,), CAP - 1, jnp.int32)
        p1 = plsc.load_gather(hpos_v, [top])
        d1 = plsc.load_gather(hdst_v, [top])
        base = CAP - cnt1 - (L - rem1)
        msk = iota >= (L - rem1)
        hpos_v[pl.ds(base, L)] = jnp.where(msk, hpos_v[pl.ds(base, L)], p1)
        hdst_v[pl.ds(base, L)] = jnp.where(msk, hdst_v[pl.ds(base, L)], d1)

    nv0 = lax.div(cnt0 + jnp.int32(L - 1), jnp.int32(L))
    nv1 = lax.div(cnt1 + jnp.int32(L - 1), jnp.int32(L))
    h1start = CAP - nv1 * L

    def _patch(vbase, h):
        def pbody(j, carry):
            base = vbase + j * L
            pos = hpos_v[pl.ds(base, L)]
            dst = hdst_v[pl.ds(base, L)]
            pt = jnp.right_shift(pos, 7)
            pi = jnp.bitwise_and(pos, 127)
            dtl = jnp.right_shift(dst, 7) - h * HALF
            di = jnp.bitwise_and(dst, 127)
            for s in range(8):
                fs = jnp.full((L,), s, jnp.int32)
                val = plsc.load_gather(bfb_v, [zero, pt, fs, pi])
                plsc.store_scatter(blk_v, [zero, dtl, fs, di], val)
            return carry

        return pbody

    for k in range(NG_PER):
        g = wid + k * NW

        @pl.when(g < G)
        def _unit(g=g):
            pltpu.sync_copy(batch_hbm.at[pl.ds(g, 1)], bfb_v)
            for h in range(2):
                pltpu.sync_copy(
                    feat_hbm.at[pl.ds(g, 1), pl.ds(h * HALF, HALF)], blk_v)
                nv = nv0 if h == 0 else nv1
                vbase = 0 if h == 0 else h1start
                lax.fori_loop(0, nv, _patch(vbase, h), 0)
                pltpu.sync_copy(
                    blk_v, out_hbm.at[pl.ds(g, 1), pl.ds(h * HALF, HALF)])


_sc_call = functools.partial(
    pl.kernel,
    out_type=jax.ShapeDtypeStruct((G, QT, 8, 128), jnp.float32),
    mesh=plsc.VectorSubcoreMesh(core_axis_name="c", subcore_axis_name="s"),
    compiler_params=pltpu.CompilerParams(needs_layout_passes=False),
    scratch_types=[
        pltpu.VMEM((BATCH,), jnp.int32),
        pltpu.VMEM((QUEUE,), jnp.int32),
        pltpu.VMEM((CAP,), jnp.int32),
        pltpu.VMEM((CAP,), jnp.int32),
        pltpu.VMEM((1, HALF, 8, 128), jnp.float32),
        pltpu.VMEM((1, PT, 8, 128), jnp.float32),
    ],
)(_sc_body)


def kernel(batch_features, batch_indices, features):
    bf = (batch_features.transpose(2, 3, 1, 0)
          .reshape(7, 7, 8, 8, PT, 128).transpose(0, 1, 2, 4, 3, 5)
          .reshape(G, PT, 8, 128))
    ft = (features.transpose(2, 3, 1, 0)
          .reshape(7, 7, 8, 8, QT, 128).transpose(0, 1, 2, 4, 3, 5)
          .reshape(G, QT, 8, 128))
    out = _sc_call(bf, batch_indices, ft)
    return (out.reshape(7, 7, 8, QT, 8, 128).transpose(0, 1, 2, 4, 3, 5)
            .reshape(7, 7, 64, QUEUE).transpose(3, 2, 0, 1))
